# Initial kernel scaffold; baseline (speedup 1.0000x reference)
import jax
import jax.numpy as jnp
from jax.experimental import pallas as pl


def kernel(u, neighs, features):
    raise NotImplementedError("write your pallas kernel here")


# speedup vs baseline: 11.4939x; 11.4939x over previous
import functools

import jax
import jax.numpy as jnp
from jax import lax
from jax.experimental import pallas as pl
from jax.experimental.pallas import tpu as pltpu
from jax.experimental.pallas import tpu_sc as plsc

E = 500000
N = 100000
D = 128
NC, NS, L = 2, 16, 16
NW = NC * NS
ROWS = 123
CHUNK = ROWS * 128
EP = NW * CHUNK
PAD_IDX = N
H = 100352
HS = H // NS
C = 1024
G = H // C


_sc_mesh = plsc.VectorSubcoreMesh(
    core_axis_name="c", subcore_axis_name="s", num_cores=NC, num_subcores=NS)


@functools.partial(
    pl.kernel,
    out_type=jax.ShapeDtypeStruct((NC, H), jnp.float32),
    mesh=_sc_mesh,
    scratch_types=[
        pltpu.VMEM((CHUNK,), jnp.int32),
        pltpu.VMEM((CHUNK,), jnp.float32),
        pltpu.VMEM((HS,), jnp.float32),
        pltpu.VMEM_SHARED((H,), jnp.float32),
    ],
    compiler_params=pltpu.CompilerParams(needs_layout_passes=False),
)
def _sc_hist(neighs_hbm, out_hbm, idx_v, ones_v, zbuf_v, hist_sh):
    cid = lax.axis_index("c")
    sid = lax.axis_index("s")
    wid = sid * NC + cid

    pltpu.sync_copy(neighs_hbm.at[wid], idx_v)

    ones = jnp.full((L,), 1.0, jnp.float32)
    zeros = jnp.zeros((L,), jnp.float32)

    def fill_body(j, _):
        ones_v[pl.ds(j * L, L)] = ones
        return 0

    lax.fori_loop(0, CHUNK // L, fill_body, 0, unroll=False)

    def zero_body(i, _):
        zbuf_v[pl.ds(i * L, L)] = zeros
        return 0

    lax.fori_loop(0, HS // L, zero_body, 0, unroll=False)
    pltpu.sync_copy(zbuf_v, hist_sh.at[pl.ds(sid * HS, HS)])
    plsc.subcore_barrier()

    pltpu.sync_copy(ones_v, hist_sh.at[idx_v], add=True)
    plsc.subcore_barrier()

    @pl.when(sid == 0)
    def _writeback():
        pltpu.sync_copy(hist_sh, out_hbm.at[cid])


def _tc_body(hist_ref, feat_ref, out_ref):
    g = pl.program_id(0)

    @pl.when(g == 0)
    def _init():
        out_ref[...] = jnp.zeros_like(out_ref)

    c = jnp.sum(hist_ref[...], axis=0, keepdims=True)
    f = feat_ref[...]
    row = g * C + lax.broadcasted_iota(jnp.int32, (C, D), 0)
    f = jnp.where(row < N, f, 0.0)
    out_ref[...] += lax.dot_general(
        c, f, (((1,), (0,)), ((), ())), preferred_element_type=jnp.float32)

    @pl.when(g == G - 1)
    def _scale():
        out_ref[...] *= jnp.float32(1.0 / E)


_tc_matvec = pl.pallas_call(
    _tc_body,
    grid=(G,),
    in_specs=[
        pl.BlockSpec((NC, C), lambda g: (0, g)),
        pl.BlockSpec((C, D), lambda g: (g, 0)),
    ],
    out_specs=pl.BlockSpec((1, D), lambda g: (0, 0)),
    out_shape=jax.ShapeDtypeStruct((1, D), jnp.float32),
)


def kernel(u, neighs, features):
    idx = jnp.concatenate(
        [neighs.astype(jnp.int32),
         jnp.full((EP - E,), PAD_IDX, jnp.int32)]).reshape(NW, CHUNK)
    hist = _sc_hist(idx)
    out = _tc_matvec(hist, features)
    return out[0]


# speedup vs baseline: 18.3618x; 1.5975x over previous
import functools

import jax
import jax.numpy as jnp
from jax import lax
from jax.experimental import pallas as pl
from jax.experimental.pallas import tpu as pltpu
from jax.experimental.pallas import tpu_sc as plsc

E = 500000
N = 100000
D = 128
NC, NS, L = 2, 16, 16
NW = NC * NS
ROWS = 123
CHUNK = ROWS * 128
EP = NW * CHUNK
PAD_IDX = N
H = 102400
HS = H // NS
C = 4096
G = H // C


_sc_mesh = plsc.VectorSubcoreMesh(
    core_axis_name="c", subcore_axis_name="s", num_cores=NC, num_subcores=NS)


@functools.partial(
    pl.kernel,
    out_type=jax.ShapeDtypeStruct((NC, H), jnp.float32),
    mesh=_sc_mesh,
    scratch_types=[
        pltpu.VMEM((CHUNK,), jnp.int32),
        pltpu.VMEM((CHUNK,), jnp.float32),
        pltpu.VMEM((HS,), jnp.float32),
        pltpu.VMEM_SHARED((H,), jnp.float32),
    ],
    compiler_params=pltpu.CompilerParams(needs_layout_passes=False),
)
def _sc_hist(neighs_hbm, out_hbm, idx_v, ones_v, zbuf_v, hist_sh):
    cid = lax.axis_index("c")
    sid = lax.axis_index("s")
    wid = sid * NC + cid

    pltpu.sync_copy(neighs_hbm.at[wid], idx_v)

    ones = jnp.full((L,), 1.0, jnp.float32)
    zeros = jnp.zeros((L,), jnp.float32)

    def fill_body(j, _):
        ones_v[pl.ds(j * L, L)] = ones
        return 0

    lax.fori_loop(0, CHUNK // L, fill_body, 0, unroll=False)

    def zero_body(i, _):
        zbuf_v[pl.ds(i * L, L)] = zeros
        return 0

    lax.fori_loop(0, HS // L, zero_body, 0, unroll=False)
    pltpu.sync_copy(zbuf_v, hist_sh.at[pl.ds(sid * HS, HS)])
    plsc.subcore_barrier()

    pltpu.sync_copy(ones_v, hist_sh.at[idx_v], add=True)
    plsc.subcore_barrier()

    @pl.when(sid == 0)
    def _writeback():
        pltpu.sync_copy(hist_sh, out_hbm.at[cid])


def _tc_body(hist_ref, feat_ref, out_ref):
    g = pl.program_id(0)

    @pl.when(g == 0)
    def _init():
        out_ref[...] = jnp.zeros_like(out_ref)

    c = jnp.sum(hist_ref[...], axis=0, keepdims=True)
    f = feat_ref[...]
    row = g * C + lax.broadcasted_iota(jnp.int32, (C, D), 0)
    f = jnp.where(row < N, f, 0.0)
    out_ref[...] += lax.dot_general(
        c, f, (((1,), (0,)), ((), ())), preferred_element_type=jnp.float32)

    @pl.when(g == G - 1)
    def _scale():
        out_ref[...] *= jnp.float32(1.0 / E)


_tc_matvec = pl.pallas_call(
    _tc_body,
    grid=(G,),
    in_specs=[
        pl.BlockSpec((NC, C), lambda g: (0, g)),
        pl.BlockSpec((C, D), lambda g: (g, 0)),
    ],
    out_specs=pl.BlockSpec((1, D), lambda g: (0, 0)),
    out_shape=jax.ShapeDtypeStruct((1, D), jnp.float32),
)


def kernel(u, neighs, features):
    idx = jnp.concatenate(
        [neighs.astype(jnp.int32),
         jnp.full((EP - E,), PAD_IDX, jnp.int32)]).reshape(NW, CHUNK)
    hist = _sc_hist(idx)
    out = _tc_matvec(hist, features)
    return out[0]


# speedup vs baseline: 21.3830x; 1.1645x over previous
import functools

import jax
import jax.numpy as jnp
from jax import lax
from jax.experimental import pallas as pl
from jax.experimental.pallas import tpu as pltpu
from jax.experimental.pallas import tpu_sc as plsc

E = 500000
N = 100000
D = 128
NC, NS, L = 2, 16, 16
NW = NC * NS
ROWS = 123
CHUNK = ROWS * 128
EP = NW * CHUNK
PAD_IDX = N
H = 102400
HS = H // NS
C = 12800
G = H // C


_sc_mesh = plsc.VectorSubcoreMesh(
    core_axis_name="c", subcore_axis_name="s", num_cores=NC, num_subcores=NS)


@functools.partial(
    pl.kernel,
    out_type=jax.ShapeDtypeStruct((NC, H), jnp.float32),
    mesh=_sc_mesh,
    scratch_types=[
        pltpu.VMEM((CHUNK,), jnp.int32),
        pltpu.VMEM((CHUNK,), jnp.float32),
        pltpu.VMEM((HS,), jnp.float32),
        pltpu.VMEM_SHARED((H,), jnp.float32),
    ],
    compiler_params=pltpu.CompilerParams(needs_layout_passes=False),
)
def _sc_hist(neighs_hbm, out_hbm, idx_v, ones_v, zbuf_v, hist_sh):
    cid = lax.axis_index("c")
    sid = lax.axis_index("s")
    wid = sid * NC + cid

    pltpu.sync_copy(neighs_hbm.at[wid], idx_v)

    ones = jnp.full((L,), 1.0, jnp.float32)
    zeros = jnp.zeros((L,), jnp.float32)

    def fill_body(j, _):
        ones_v[pl.ds(j * L, L)] = ones
        return 0

    lax.fori_loop(0, CHUNK // L, fill_body, 0, unroll=False)

    def zero_body(i, _):
        zbuf_v[pl.ds(i * L, L)] = zeros
        return 0

    lax.fori_loop(0, HS // L, zero_body, 0, unroll=False)
    pltpu.sync_copy(zbuf_v, hist_sh.at[pl.ds(sid * HS, HS)])
    plsc.subcore_barrier()

    pltpu.sync_copy(ones_v, hist_sh.at[idx_v], add=True)
    plsc.subcore_barrier()

    @pl.when(sid == 0)
    def _writeback():
        pltpu.sync_copy(hist_sh, out_hbm.at[cid])


def _tc_body(hist_ref, feat_ref, out_ref):
    g = pl.program_id(0)

    @pl.when(g == 0)
    def _init():
        out_ref[...] = jnp.zeros_like(out_ref)

    c = jnp.sum(hist_ref[...], axis=0, keepdims=True)
    f = feat_ref[...]
    row = g * C + lax.broadcasted_iota(jnp.int32, (C, D), 0)
    f = jnp.where(row < N, f, 0.0)
    out_ref[...] += lax.dot_general(
        c, f, (((1,), (0,)), ((), ())), preferred_element_type=jnp.float32)

    @pl.when(g == G - 1)
    def _scale():
        out_ref[...] *= jnp.float32(1.0 / E)


_tc_matvec = pl.pallas_call(
    _tc_body,
    grid=(G,),
    in_specs=[
        pl.BlockSpec((NC, C), lambda g: (0, g)),
        pl.BlockSpec((C, D), lambda g: (g, 0)),
    ],
    out_specs=pl.BlockSpec((1, D), lambda g: (0, 0)),
    out_shape=jax.ShapeDtypeStruct((1, D), jnp.float32),
)


def kernel(u, neighs, features):
    idx = jnp.concatenate(
        [neighs.astype(jnp.int32),
         jnp.full((EP - E,), PAD_IDX, jnp.int32)]).reshape(NW, CHUNK)
    hist = _sc_hist(idx)
    out = _tc_matvec(hist, features)
    return out[0]
